# Initial kernel scaffold; baseline (speedup 1.0000x reference)
import jax
import jax.numpy as jnp
from jax.experimental import pallas as pl


def kernel(E_real, E_imag, batch_y, k_all, node_batch, A_rows, A_cols, A_vals_real, A_vals_imag, b_real, b_imag):
    raise NotImplementedError("write your pallas kernel here")


# speedup vs baseline: 110.5520x; 110.5520x over previous
import functools

import jax
import jax.numpy as jnp
from jax import lax
from jax.experimental import pallas as pl
from jax.experimental.pallas import tpu as pltpu
from jax.experimental.pallas import tpu_sc as plsc

B = 4
NP = 10000
NNZ = 160000
N = B * NP

NC = 2
NS = 16
L = 16

CHUNK = NNZ // NS
NP_PAD = 10240
SLICE = NP_PAD // NS
N_PAD = 40960
MSE_CHUNK = N_PAD // (NC * NS)


def _sc_body(er_hbm, ei_hbm, rows_hbm, cols_hbm, vr_hbm, vi_hbm,
             bpr_hbm, bpi_hbm, emr_hbm, emi_hbm, ymr_hbm, ymi_hbm,
             out_hbm,
             rowv, colv, vrv, viv, xr, xi, accr, acci,
             tmp16, brv, biv, m0, m1, m2, m3, outv, shared):
  c = lax.axis_index("c")
  s = lax.axis_index("s")

  zeros = jnp.zeros((L,), jnp.float32)
  loss_vec = zeros

  for b_local in range(2):
    bi = 2 * c + b_local

    nnz_base = bi * NNZ + s * CHUNK
    pltpu.sync_copy(rows_hbm.at[pl.ds(nnz_base, CHUNK)], rowv)
    pltpu.sync_copy(cols_hbm.at[pl.ds(nnz_base, CHUNK)], colv)
    pltpu.sync_copy(vr_hbm.at[pl.ds(nnz_base, CHUNK)], vrv)
    pltpu.sync_copy(vi_hbm.at[pl.ds(nnz_base, CHUNK)], viv)
    pltpu.sync_copy(er_hbm.at[pl.ds(bi * NP, NP)], xr)
    pltpu.sync_copy(ei_hbm.at[pl.ds(bi * NP, NP)], xi)

    def zero_body(k, _):
      off = pl.ds(k * L, L)
      accr[off] = zeros
      acci[off] = zeros
      return 0
    lax.fori_loop(0, NP_PAD // L, zero_body, 0)

    def nnz_body(j, _):
      off = pl.ds(j * L, L)
      cols16 = colv[off]
      xcr = plsc.load_gather(xr, [cols16])
      xci = plsc.load_gather(xi, [cols16])
      wr = vrv[off]
      wi = viv[off]
      ar = wr * xcr - wi * xci
      ai = wr * xci + wi * xcr
      r16 = rowv[off]
      plsc.addupdate_scatter(accr, [r16], ar)
      plsc.addupdate_scatter(acci, [r16], ai)
      return 0
    lax.fori_loop(0, CHUNK // L, nnz_body, 0)

    pltpu.sync_copy(accr, shared.at[s, 0])
    pltpu.sync_copy(acci, shared.at[s, 1])
    plsc.subcore_barrier()

    row_base = s * SLICE
    pltpu.sync_copy(shared.at[:, :, pl.ds(row_base, SLICE)], tmp16)
    pltpu.sync_copy(bpr_hbm.at[pl.ds(bi * NP_PAD + row_base, SLICE)], brv)
    pltpu.sync_copy(bpi_hbm.at[pl.ds(bi * NP_PAD + row_base, SLICE)], biv)

    def res_body(k, acc):
      off = pl.ds(k * L, L)
      axr = tmp16[0, 0, off]
      axi = tmp16[0, 1, off]
      for t in range(1, NS):
        axr = axr + tmp16[t, 0, off]
        axi = axi + tmp16[t, 1, off]
      rr = brv[off] - axr
      ri = biv[off] - axi
      return acc + rr * rr + ri * ri
    loss_vec = lax.fori_loop(0, SLICE // L, res_body, loss_vec)

    plsc.subcore_barrier()

  wid = c * NS + s
  mse_base = wid * MSE_CHUNK
  pltpu.sync_copy(emr_hbm.at[pl.ds(mse_base, MSE_CHUNK)], m0)
  pltpu.sync_copy(emi_hbm.at[pl.ds(mse_base, MSE_CHUNK)], m1)
  pltpu.sync_copy(ymr_hbm.at[pl.ds(mse_base, MSE_CHUNK)], m2)
  pltpu.sync_copy(ymi_hbm.at[pl.ds(mse_base, MSE_CHUNK)], m3)

  def mse_body(k, acc):
    off = pl.ds(k * L, L)
    dr = m0[off] - m2[off]
    di = m1[off] - m3[off]
    return acc + dr * dr + di * di
  loss_vec = lax.fori_loop(0, MSE_CHUNK // L, mse_body, loss_vec)

  outv[...] = loss_vec * jnp.float32(0.5 / N)
  pltpu.sync_copy(outv, out_hbm.at[wid])


@jax.jit
def _run(er, ei, rows, cols, vr, vi, bpr, bpi, emr, emi, ymr, ymi):
  mesh = plsc.VectorSubcoreMesh(
      core_axis_name="c", subcore_axis_name="s",
      num_cores=NC, num_subcores=NS)
  f = pl.kernel(
      _sc_body,
      out_type=jax.ShapeDtypeStruct((NC * NS, L), jnp.float32),
      mesh=mesh,
      compiler_params=pltpu.CompilerParams(needs_layout_passes=False),
      scratch_types=[
          pltpu.VMEM((CHUNK,), jnp.int32),
          pltpu.VMEM((CHUNK,), jnp.int32),
          pltpu.VMEM((CHUNK,), jnp.float32),
          pltpu.VMEM((CHUNK,), jnp.float32),
          pltpu.VMEM((NP,), jnp.float32),
          pltpu.VMEM((NP,), jnp.float32),
          pltpu.VMEM((NP_PAD,), jnp.float32),
          pltpu.VMEM((NP_PAD,), jnp.float32),
          pltpu.VMEM((NS, 2, SLICE), jnp.float32),
          pltpu.VMEM((SLICE,), jnp.float32),
          pltpu.VMEM((SLICE,), jnp.float32),
          pltpu.VMEM((MSE_CHUNK,), jnp.float32),
          pltpu.VMEM((MSE_CHUNK,), jnp.float32),
          pltpu.VMEM((MSE_CHUNK,), jnp.float32),
          pltpu.VMEM((MSE_CHUNK,), jnp.float32),
          pltpu.VMEM((L,), jnp.float32),
          pltpu.VMEM_SHARED((NS, 2, NP_PAD), jnp.float32),
      ],
  )
  return f(er, ei, rows, cols, vr, vi, bpr, bpi, emr, emi, ymr, ymi)


def kernel(E_real, E_imag, batch_y, k_all, node_batch, A_rows, A_cols,
           A_vals_real, A_vals_imag, b_real, b_imag):
  del k_all, node_batch

  rows = A_rows.reshape(-1)
  cols = A_cols.reshape(-1)
  vr = A_vals_real.reshape(-1)
  vi = A_vals_imag.reshape(-1)

  bpr = jnp.zeros((B, NP_PAD), jnp.float32).at[:, :NP].set(b_real).reshape(-1)
  bpi = jnp.zeros((B, NP_PAD), jnp.float32).at[:, :NP].set(b_imag).reshape(-1)

  emr = jnp.zeros((N_PAD,), jnp.float32).at[:N].set(E_real)
  emi = jnp.zeros((N_PAD,), jnp.float32).at[:N].set(E_imag)
  ymr = jnp.zeros((N_PAD,), jnp.float32).at[:N].set(batch_y[:, 0])
  ymi = jnp.zeros((N_PAD,), jnp.float32).at[:N].set(batch_y[:, 1])

  partials = _run(E_real, E_imag, rows, cols, vr, vi,
                  bpr, bpi, emr, emi, ymr, ymi)
  return jnp.sum(partials)


# speedup vs baseline: 132.6019x; 1.1995x over previous
import functools

import jax
import jax.numpy as jnp
from jax import lax
from jax.experimental import pallas as pl
from jax.experimental.pallas import tpu as pltpu
from jax.experimental.pallas import tpu_sc as plsc

B = 4
NP = 10000
NNZ = 160000
N = B * NP

NC = 2
NS = 16
L = 16

CHUNK = NNZ // NS
NP_PAD = 10240
SLICE = NP_PAD // NS
N_PAD = 40960
MSE_CHUNK = N_PAD // (NC * NS)
U = 5


def _sc_body(er_hbm, ei_hbm, rows_hbm, cols_hbm, vr_hbm, vi_hbm,
             bpr_hbm, bpi_hbm, emr_hbm, emi_hbm, ymr_hbm, ymi_hbm,
             out_hbm,
             rowv, colv, vrv, viv, xr, xi, accr, acci,
             tmp16, brv, biv, m0, m1, m2, m3, outv, shared,
             sem_chunk, sem_x, sem_b, sem_mse):
  c = lax.axis_index("c")
  s = lax.axis_index("s")

  zeros = jnp.zeros((L,), jnp.float32)
  loss_vec = zeros
  wid = c * NS + s

  mse_base = wid * MSE_CHUNK
  mse_copies = [
      pltpu.async_copy(emr_hbm.at[pl.ds(mse_base, MSE_CHUNK)], m0, sem_mse),
      pltpu.async_copy(emi_hbm.at[pl.ds(mse_base, MSE_CHUNK)], m1, sem_mse),
      pltpu.async_copy(ymr_hbm.at[pl.ds(mse_base, MSE_CHUNK)], m2, sem_mse),
      pltpu.async_copy(ymi_hbm.at[pl.ds(mse_base, MSE_CHUNK)], m3, sem_mse),
  ]

  row_base = s * SLICE

  for b_local in range(2):
    bi = 2 * c + b_local

    nnz_base = bi * NNZ + s * CHUNK
    chunk_copies = [
        pltpu.async_copy(rows_hbm.at[pl.ds(nnz_base, CHUNK)], rowv, sem_chunk),
        pltpu.async_copy(cols_hbm.at[pl.ds(nnz_base, CHUNK)], colv, sem_chunk),
        pltpu.async_copy(vr_hbm.at[pl.ds(nnz_base, CHUNK)], vrv, sem_chunk),
        pltpu.async_copy(vi_hbm.at[pl.ds(nnz_base, CHUNK)], viv, sem_chunk),
    ]
    x_copies = [
        pltpu.async_copy(er_hbm.at[pl.ds(bi * NP, NP)], xr, sem_x),
        pltpu.async_copy(ei_hbm.at[pl.ds(bi * NP, NP)], xi, sem_x),
    ]
    b_copies = [
        pltpu.async_copy(
            bpr_hbm.at[pl.ds(bi * NP_PAD + row_base, SLICE)], brv, sem_b),
        pltpu.async_copy(
            bpi_hbm.at[pl.ds(bi * NP_PAD + row_base, SLICE)], biv, sem_b),
    ]

    def zero_body(k, _):
      for u in range(4):
        off = pl.ds(k * (4 * L) + u * L, L)
        accr[off] = zeros
        acci[off] = zeros
      return 0
    lax.fori_loop(0, NP_PAD // (4 * L), zero_body, 0)

    for cp in chunk_copies:
      cp.wait()
    for cp in x_copies:
      cp.wait()

    def nnz_body(j, _):
      base = j * (U * L)
      for u in range(U):
        off = pl.ds(base + u * L, L)
        cols16 = colv[off]
        xcr = plsc.load_gather(xr, [cols16])
        xci = plsc.load_gather(xi, [cols16])
        wr = vrv[off]
        wi = viv[off]
        ar = wr * xcr - wi * xci
        ai = wr * xci + wi * xcr
        r16 = rowv[off]
        plsc.addupdate_scatter(accr, [r16], ar)
        plsc.addupdate_scatter(acci, [r16], ai)
      return 0
    lax.fori_loop(0, CHUNK // (U * L), nnz_body, 0)

    pltpu.sync_copy(accr, shared.at[s, 0])
    pltpu.sync_copy(acci, shared.at[s, 1])
    plsc.subcore_barrier()

    pltpu.sync_copy(shared.at[:, :, pl.ds(row_base, SLICE)], tmp16)
    for cp in b_copies:
      cp.wait()

    def res_body(k, acc):
      off = pl.ds(k * L, L)
      axr = tmp16[0, 0, off]
      axi = tmp16[0, 1, off]
      for t in range(1, NS):
        axr = axr + tmp16[t, 0, off]
        axi = axi + tmp16[t, 1, off]
      rr = brv[off] - axr
      ri = biv[off] - axi
      return acc + rr * rr + ri * ri
    loss_vec = lax.fori_loop(0, SLICE // L, res_body, loss_vec)

    plsc.subcore_barrier()

  for cp in mse_copies:
    cp.wait()

  def mse_body(k, acc):
    off = pl.ds(k * L, L)
    dr = m0[off] - m2[off]
    di = m1[off] - m3[off]
    return acc + dr * dr + di * di
  loss_vec = lax.fori_loop(0, MSE_CHUNK // L, mse_body, loss_vec)

  outv[...] = loss_vec * jnp.float32(0.5 / N)
  pltpu.sync_copy(outv, out_hbm.at[wid])


@jax.jit
def _run(er, ei, rows, cols, vr, vi, bpr, bpi, emr, emi, ymr, ymi):
  mesh = plsc.VectorSubcoreMesh(
      core_axis_name="c", subcore_axis_name="s",
      num_cores=NC, num_subcores=NS)
  f = pl.kernel(
      _sc_body,
      out_type=jax.ShapeDtypeStruct((NC * NS, L), jnp.float32),
      mesh=mesh,
      compiler_params=pltpu.CompilerParams(needs_layout_passes=False),
      scratch_types=[
          pltpu.VMEM((CHUNK,), jnp.int32),
          pltpu.VMEM((CHUNK,), jnp.int32),
          pltpu.VMEM((CHUNK,), jnp.float32),
          pltpu.VMEM((CHUNK,), jnp.float32),
          pltpu.VMEM((NP,), jnp.float32),
          pltpu.VMEM((NP,), jnp.float32),
          pltpu.VMEM((NP_PAD,), jnp.float32),
          pltpu.VMEM((NP_PAD,), jnp.float32),
          pltpu.VMEM((NS, 2, SLICE), jnp.float32),
          pltpu.VMEM((SLICE,), jnp.float32),
          pltpu.VMEM((SLICE,), jnp.float32),
          pltpu.VMEM((MSE_CHUNK,), jnp.float32),
          pltpu.VMEM((MSE_CHUNK,), jnp.float32),
          pltpu.VMEM((MSE_CHUNK,), jnp.float32),
          pltpu.VMEM((MSE_CHUNK,), jnp.float32),
          pltpu.VMEM((L,), jnp.float32),
          pltpu.VMEM_SHARED((NS, 2, NP_PAD), jnp.float32),
          pltpu.SemaphoreType.DMA,
          pltpu.SemaphoreType.DMA,
          pltpu.SemaphoreType.DMA,
          pltpu.SemaphoreType.DMA,
      ],
  )
  return f(er, ei, rows, cols, vr, vi, bpr, bpi, emr, emi, ymr, ymi)


def kernel(E_real, E_imag, batch_y, k_all, node_batch, A_rows, A_cols,
           A_vals_real, A_vals_imag, b_real, b_imag):
  del k_all, node_batch

  rows = A_rows.reshape(-1)
  cols = A_cols.reshape(-1)
  vr = A_vals_real.reshape(-1)
  vi = A_vals_imag.reshape(-1)

  bpr = jnp.zeros((B, NP_PAD), jnp.float32).at[:, :NP].set(b_real).reshape(-1)
  bpi = jnp.zeros((B, NP_PAD), jnp.float32).at[:, :NP].set(b_imag).reshape(-1)

  emr = jnp.zeros((N_PAD,), jnp.float32).at[:N].set(E_real)
  emi = jnp.zeros((N_PAD,), jnp.float32).at[:N].set(E_imag)
  ymr = jnp.zeros((N_PAD,), jnp.float32).at[:N].set(batch_y[:, 0])
  ymi = jnp.zeros((N_PAD,), jnp.float32).at[:N].set(batch_y[:, 1])

  partials = _run(E_real, E_imag, rows, cols, vr, vi,
                  bpr, bpi, emr, emi, ymr, ymi)
  return jnp.sum(partials)
